# Initial kernel scaffold; baseline (speedup 1.0000x reference)
import jax
import jax.numpy as jnp
from jax.experimental import pallas as pl


def kernel(input, rois):
    raise NotImplementedError("write your pallas kernel here")


# speedup vs baseline: 11.8610x; 11.8610x over previous
import functools

import jax
import jax.numpy as jnp
from jax import lax
from jax.experimental import pallas as pl
from jax.experimental.pallas import tpu as pltpu

_PH, _PW = 7, 7
_SCALE = 1.0
_NEG = float("-inf")


def _roi_body(meta_ref, x_ref, o_ref, *, H, W, C, max_span):
    i = pl.program_id(0)
    hs, he, ws, we = [], [], [], []
    for j in range(_PH):
        word = meta_ref[i, 2 + j]
        hs.append(word & 0xFF)
        he.append((word >> 8) & 0xFF)
        ws.append((word >> 16) & 0xFF)
        we.append((word >> 24) & 0xFF)

    wcoord = lax.broadcasted_iota(jnp.int32, (W, 1), 0)

    v1 = []
    for ph in range(_PH):
        base = jnp.minimum(hs[ph], H - max_span)
        rows = x_ref[0, pl.ds(base, max_span)]
        acc = jnp.full((W, C), _NEG, jnp.float32)
        for d in range(max_span):
            take = (base + d >= hs[ph]) & (base + d < he[ph])
            acc = jnp.where(take, jnp.maximum(acc, rows[d]), acc)
        v1.append(acc)

    out_rows = []
    for ph in range(_PH):
        h_empty = he[ph] <= hs[ph]
        for pw in range(_PW):
            mask = (wcoord >= ws[pw]) & (wcoord < we[pw])
            val = jnp.max(jnp.where(mask, v1[ph], _NEG), axis=0, keepdims=True)
            empty = h_empty | (we[pw] <= ws[pw])
            out_rows.append(jnp.where(empty, 0.0, val))
    o_ref[0] = jnp.concatenate(out_rows, axis=0)


def _bin_bounds(start, end, nbins, dim):
    length = jnp.maximum(end - start + 1.0, 1.0)
    bsz = length / nbins
    p = jnp.arange(nbins, dtype=jnp.float32)
    lo = jnp.clip(jnp.floor(p[None, :] * bsz[:, None]) + start[:, None], 0.0, float(dim))
    hi = jnp.clip(jnp.ceil((p[None, :] + 1.0) * bsz[:, None]) + start[:, None], 0.0, float(dim))
    return lo.astype(jnp.int32), hi.astype(jnp.int32)


def kernel(input, rois):
    N, C, H, W = input.shape
    M = rois.shape[0]
    max_span = -(-W // _PW) + 2

    xt = jnp.transpose(input, (0, 2, 3, 1))

    b = rois[:, 0].astype(jnp.int32)
    start_w = jnp.round(rois[:, 1] * _SCALE)
    start_h = jnp.round(rois[:, 2] * _SCALE)
    end_w = jnp.round(rois[:, 3] * _SCALE)
    end_h = jnp.round(rois[:, 4] * _SCALE)
    hs, he = _bin_bounds(start_h, end_h, _PH, H)
    ws, we = _bin_bounds(start_w, end_w, _PW, W)
    packed = hs | (he << 8) | (ws << 16) | (we << 24)

    order = jnp.argsort(b)
    meta = jnp.concatenate(
        [b[order][:, None], order[:, None].astype(jnp.int32), packed[order]],
        axis=1)

    grid_spec = pltpu.PrefetchScalarGridSpec(
        num_scalar_prefetch=1,
        grid=(M,),
        in_specs=[
            pl.BlockSpec((1, H, W, C), lambda i, m: (m[i, 0], 0, 0, 0)),
        ],
        out_specs=pl.BlockSpec((1, _PH * _PW, C), lambda i, m: (m[i, 1], 0, 0)),
    )
    out = pl.pallas_call(
        functools.partial(_roi_body, H=H, W=W, C=C, max_span=max_span),
        grid_spec=grid_spec,
        out_shape=jax.ShapeDtypeStruct((M, _PH * _PW, C), jnp.float32),
    )(meta, xt)
    return out.transpose(0, 2, 1).reshape(M, C, _PH, _PW)
